# Initial kernel scaffold; baseline (speedup 1.0000x reference)
import jax
import jax.numpy as jnp
from jax.experimental import pallas as pl


def kernel(tokens, emb):
    raise NotImplementedError("write your pallas kernel here")


# speedup vs baseline: 2.3652x; 2.3652x over previous
import functools

import jax
import jax.numpy as jnp
from jax import lax
from jax.experimental import pallas as pl
from jax.experimental.pallas import tpu as pltpu
from jax.experimental.pallas import tpu_sc as plsc


def _norm_body(x_ref, o_ref):
    x = x_ref[...]
    s = jnp.sum(x * x, axis=1, keepdims=True)
    o_ref[...] = x / jnp.sqrt(s)


def _normalize_table(emb):
    v, d = emb.shape
    r = 512
    return pl.pallas_call(
        _norm_body,
        grid=(pl.cdiv(v, r),),
        in_specs=[pl.BlockSpec((r, d), lambda i: (i, 0))],
        out_specs=pl.BlockSpec((r, d), lambda i: (i, 0)),
        out_shape=jax.ShapeDtypeStruct((v, d), jnp.float32),
    )(emb)


_NC, _NS = 2, 16
_NW = _NC * _NS
_CHUNK = 64


_PHASES = 2


def _make_sc_gather(b_total, d):
    bpw = b_total // _NW
    n_chunks = bpw // _CHUNK
    cpp = n_chunks // _PHASES
    assert cpp % 2 == 0

    mesh = plsc.VectorSubcoreMesh(core_axis_name="c", subcore_axis_name="s")

    @functools.partial(
        pl.kernel,
        mesh=mesh,
        out_type=jax.ShapeDtypeStruct((b_total, d), jnp.float32),
        scratch_types=[
            pltpu.VMEM((cpp, _CHUNK), jnp.int32),
            pltpu.VMEM((_CHUNK, d), jnp.float32),
            pltpu.VMEM((_CHUNK, d), jnp.float32),
            pltpu.SemaphoreType.DMA,
            pltpu.SemaphoreType.DMA,
        ],
    )
    def sc_gather(table_hbm, idx_hbm, out_hbm, idx_v, rows0, rows1, sem0, sem1):
        wid = lax.axis_index("s") * _NC + lax.axis_index("c")
        base = wid * bpw
        row_bufs = (rows0, rows1)
        sems = (sem0, sem1)

        def run_phase(ph):
            pltpu.sync_copy(idx_hbm.at[wid, ph], idx_v)
            out0 = base + ph * cpp * _CHUNK

            def start(g, buf):
                pltpu.async_copy(
                    table_hbm.at[idx_v.at[g]], row_bufs[buf], sems[buf])

            def drain(g, buf):
                pltpu.make_async_copy(
                    table_hbm.at[idx_v.at[g]], row_bufs[buf],
                    sems[buf]).wait()
                pltpu.sync_copy(
                    row_bufs[buf],
                    out_hbm.at[pl.ds(out0 + g * _CHUNK, _CHUNK)])

            start(0, 0)
            start(1, 1)

            def body(i, carry):
                g = 2 * i
                drain(g, 0)
                start(g + 2, 0)
                drain(g + 1, 1)
                start(g + 3, 1)
                return carry

            lax.fori_loop(0, cpp // 2 - 1, body, 0, unroll=False)
            drain(cpp - 2, 0)
            drain(cpp - 1, 1)

        for ph in range(_PHASES):
            run_phase(ph)

    return sc_gather


def kernel(tokens, emb):
    bsz, seq = tokens.shape
    v, d = emb.shape
    b_total = bsz * seq

    table = _normalize_table(emb)
    idx = jnp.reshape(tokens.astype(jnp.int32),
                      (_NW, _PHASES, b_total // (_NW * _PHASES * _CHUNK),
                       _CHUNK))
    out = _make_sc_gather(b_total, d)(table, idx)
    return out.reshape(bsz, seq, d)
